# baseline (speedup 1.0000x reference)
import functools

import numpy as np

import jax
import jax.numpy as jnp
from jax import lax
from jax.experimental import pallas as pl
from jax.experimental.pallas import tpu as pltpu
from jax.experimental.pallas import tpu_sc as plsc

_R = 16384
_J = 64
_S = 16
_FD = 32
_PD = 3
_TOT = _R * _S


def _gumbel_noise_packed() -> np.ndarray:
    n = _R * _J * _S
    x0 = np.zeros(n, dtype=np.uint32)
    x1 = np.arange(n, dtype=np.uint32)
    ks0 = np.uint32(0)
    ks1 = np.uint32(42)
    ks2 = np.uint32(ks0 ^ ks1 ^ np.uint32(0x1BD11BDA))
    ks = (ks0, ks1, ks2)
    rot = (13, 15, 26, 6, 17, 29, 16, 24)

    x0 += ks0
    x1 += ks1
    for g in range(5):
        for r in (rot[0:4] if g % 2 == 0 else rot[4:8]):
            x0 += x1
            x1 = ((x1 << np.uint32(r)) | (x1 >> np.uint32(32 - r)))
            x1 ^= x0
        x0 += ks[(g + 1) % 3]
        x1 += ks[(g + 2) % 3] + np.uint32(g + 1)
    bits = x0 ^ x1

    tiny = np.float32(np.finfo(np.float32).tiny)
    fb = (bits >> np.uint32(9)) | np.uint32(0x3F800000)
    f = fb.view(np.float32) - np.float32(1.0)
    u = np.maximum(tiny, f * (np.float32(1.0) - tiny) + tiny)
    g32 = (-np.log(-np.log(u))).astype(np.float32)

    g3 = g32.reshape(_R, _J, _S)
    packed = np.empty((8, _R, 128), dtype=np.float32)
    for h in (0, 1):
        packed[:, :, 64 * h:64 * h + 64] = np.ascontiguousarray(
            g3[:, :, 8 * h:8 * h + 8].transpose(2, 0, 1))
    return packed


_NOISE = _gumbel_noise_packed()

_B1 = 256
_B3 = 512
_CHUNK = 2048
_NW = 32


def _stage1_body(nr_ref, w_ref, noise_ref, px_ref, py_ref, pz_ref,
                 gi_ref, nv_ref, sp_ref):
    w = w_ref[...]
    lw = jnp.log(jnp.maximum(w, 1e-37))
    m = jnp.max(lw, axis=1, keepdims=True)
    e = jnp.exp(lw - m)
    p = e / jnp.sum(e, axis=1, keepdims=True)
    nv_all = w / (nr_ref[0, 0] * p + 1e-8)

    lw2 = jnp.concatenate([lw, lw], axis=1)
    iota_f = lax.broadcasted_iota(jnp.int32, (_B1, _J), 1).astype(jnp.float32)
    inds = [None] * _S
    for g in range(8):
        sg = noise_ref[g] + lw2
        for h in (0, 1):
            sc = sg[:, 64 * h:64 * h + 64]
            mx = jnp.max(sc, axis=1, keepdims=True)
            inds[h * 8 + g] = jnp.min(
                jnp.where(sc == mx, iota_f, float(_J)), axis=1, keepdims=True)
    ind16 = jnp.concatenate(inds, axis=1).astype(jnp.int32)

    nv16 = jnp.take_along_axis(nv_all, ind16, axis=1)
    spx = jnp.take_along_axis(px_ref[...], ind16, axis=1) * nv16
    spy = jnp.take_along_axis(py_ref[...], ind16, axis=1) * nv16
    spz = jnp.take_along_axis(pz_ref[...], ind16, axis=1) * nv16

    rows = pl.program_id(0) * _B1 + lax.broadcasted_iota(jnp.int32, (_B1, _S), 0)
    gi_ref[...] = rows * _S + (ind16 >> 2)
    nv_ref[...] = nv16
    sp_ref[...] = jnp.stack(
        [spx, spy, spz, (ind16 & 3).astype(jnp.float32)], axis=2)


def _stage1(nr, weights, noise, px, py, pz):
    return pl.pallas_call(
        _stage1_body,
        grid=(_R // _B1,),
        in_specs=[
            pl.BlockSpec(memory_space=pltpu.SMEM),
            pl.BlockSpec((_B1, _J), lambda i: (i, 0)),
            pl.BlockSpec((8, _B1, 128), lambda i: (0, i, 0)),
            pl.BlockSpec((_B1, _J), lambda i: (i, 0)),
            pl.BlockSpec((_B1, _J), lambda i: (i, 0)),
            pl.BlockSpec((_B1, _J), lambda i: (i, 0)),
        ],
        out_specs=[
            pl.BlockSpec((_B1, _S), lambda i: (i, 0)),
            pl.BlockSpec((_B1, _S), lambda i: (i, 0)),
            pl.BlockSpec((_B1, _S, _PD + 1), lambda i: (i, 0, 0)),
        ],
        out_shape=[
            jax.ShapeDtypeStruct((_R, _S), jnp.int32),
            jax.ShapeDtypeStruct((_R, _S), jnp.float32),
            jax.ShapeDtypeStruct((_R, _S, _PD + 1), jnp.float32),
        ],
    )(nr, weights, noise, px, py, pz)


_TBLK = 1024
_GCHUNK = 512


def _pack_body(x_ref, o_ref):
    ident = (lax.broadcasted_iota(jnp.int32, (128, 128), 0) ==
             lax.broadcasted_iota(jnp.int32, (128, 128), 1)).astype(jnp.float32)
    for q in range(_S):
        for p in range(_TBLK // 128):
            t = x_ref[pl.ds(128 * q, 128), pl.ds(128 * p, 128)]
            tt = lax.dot_general(t, ident, (((0,), (0,)), ((), ())),
                                 precision=lax.Precision.HIGHEST,
                                 preferred_element_type=jnp.float32)
            o_ref[pl.ds(128 * p, 128), q, :] = tt


def _pack(feat_lin):
    return pl.pallas_call(
        _pack_body,
        grid=(_R // _TBLK,),
        in_specs=[pl.BlockSpec((_J * _FD, _TBLK), lambda i: (0, i))],
        out_specs=pl.BlockSpec((_TBLK, _S, 128), lambda i: (i, 0, 0)),
        out_shape=jax.ShapeDtypeStruct((_R, _S, 128), jnp.float32),
    )(feat_lin)


@functools.cache
def _sc_gather_fn():
    @functools.partial(
        pl.kernel,
        out_type=jax.ShapeDtypeStruct((_TOT, 128), jnp.float32),
        mesh=plsc.VectorSubcoreMesh(core_axis_name="c", subcore_axis_name="s"),
        scratch_types=[
            pltpu.VMEM((_GCHUNK,), jnp.int32),
            pltpu.VMEM((_GCHUNK, 128), jnp.float32),
            pltpu.SemaphoreType.DMA,
        ],
        compiler_params=pltpu.CompilerParams(use_tc_tiling_on_sc=False),
    )
    def _sc_gather(gi_hbm, packed_hbm, outf_hbm, idx_v, fbuf, sem_f):
        wid = lax.axis_index("c") * 16 + lax.axis_index("s")
        per_w = _TOT // _NW
        for t in range(per_w // _GCHUNK):
            base = wid * per_w + t * _GCHUNK
            pltpu.sync_copy(gi_hbm.at[pl.ds(base, _GCHUNK)], idx_v)
            pltpu.async_copy(packed_hbm.at[idx_v], fbuf, sem_f).wait()
            pltpu.sync_copy(fbuf, outf_hbm.at[pl.ds(base, _GCHUNK), :])

    return _sc_gather


def _stage3_body(f_ref, sp_ref, nv_ref, o_ref):
    nv = nv_ref[...][:, :, None]
    q = sp_ref[:, :, _PD:_PD + 1].astype(jnp.int32)
    idx = q * _FD + lax.broadcasted_iota(jnp.int32, (_B3, _S, _FD), 2)
    f = jnp.take_along_axis(f_ref[...], idx, axis=2)
    o_ref[...] = jnp.concatenate([nv * f, sp_ref[:, :, 0:_PD]], axis=2)


def _stage3(outf, sp, nv):
    return pl.pallas_call(
        _stage3_body,
        grid=(_R // _B3,),
        in_specs=[
            pl.BlockSpec((_B3, _S, 128), lambda i: (i, 0, 0)),
            pl.BlockSpec((_B3, _S, _PD + 1), lambda i: (i, 0, 0)),
            pl.BlockSpec((_B3, _S), lambda i: (i, 0)),
        ],
        out_specs=pl.BlockSpec((_B3, _S, _FD + _PD), lambda i: (i, 0, 0)),
        out_shape=jax.ShapeDtypeStruct((_R, _S, _FD + _PD), jnp.float32),
    )(outf, sp, nv)


def kernel(weights, points, features, num_resample):
    nr = jnp.asarray(num_resample, jnp.float32).reshape(1, 1)
    px = points[:, :, 0]
    py = points[:, :, 1]
    pz = points[:, :, 2]
    gi, nv, sp = _stage1(nr, weights, jnp.asarray(_NOISE), px, py, pz)
    feat_lin = features.transpose(1, 2, 0).reshape(_J * _FD, _R)
    packed = _pack(feat_lin)
    outf = _sc_gather_fn()(gi.reshape(_TOT), packed.reshape(_TOT, 128))
    return _stage3(outf.reshape(_R, _S, 128), sp, nv)


# speedup vs baseline: 1.0458x; 1.0458x over previous
import functools

import numpy as np

import jax
import jax.numpy as jnp
from jax import lax
from jax.experimental import pallas as pl
from jax.experimental.pallas import tpu as pltpu
from jax.experimental.pallas import tpu_sc as plsc

_R = 16384
_J = 64
_S = 16
_FD = 32
_PD = 3
_TOT = _R * _S


def _gumbel_noise_packed() -> np.ndarray:
    n = _R * _J * _S
    x0 = np.zeros(n, dtype=np.uint32)
    x1 = np.arange(n, dtype=np.uint32)
    ks0 = np.uint32(0)
    ks1 = np.uint32(42)
    ks2 = np.uint32(ks0 ^ ks1 ^ np.uint32(0x1BD11BDA))
    ks = (ks0, ks1, ks2)
    rot = (13, 15, 26, 6, 17, 29, 16, 24)

    x0 += ks0
    x1 += ks1
    for g in range(5):
        for r in (rot[0:4] if g % 2 == 0 else rot[4:8]):
            x0 += x1
            x1 = ((x1 << np.uint32(r)) | (x1 >> np.uint32(32 - r)))
            x1 ^= x0
        x0 += ks[(g + 1) % 3]
        x1 += ks[(g + 2) % 3] + np.uint32(g + 1)
    bits = x0 ^ x1

    tiny = np.float32(np.finfo(np.float32).tiny)
    fb = (bits >> np.uint32(9)) | np.uint32(0x3F800000)
    f = fb.view(np.float32) - np.float32(1.0)
    u = np.maximum(tiny, f * (np.float32(1.0) - tiny) + tiny)
    g32 = (-np.log(-np.log(u))).astype(np.float32)

    g3 = g32.reshape(_R, _J, _S)
    packed = np.empty((8, _R, 128), dtype=np.float32)
    for h in (0, 1):
        packed[:, :, 64 * h:64 * h + 64] = np.ascontiguousarray(
            g3[:, :, 8 * h:8 * h + 8].transpose(2, 0, 1))
    return packed


_NOISE = _gumbel_noise_packed()

_B1 = 512
_B3 = 256
_CHUNK = 2048
_NW = 32


def _stage1_body(nr_ref, w_ref, noise_ref, px_ref, py_ref, pz_ref,
                 gi_ref, nv_ref, sp_ref):
    w = w_ref[...]
    lw = jnp.log(jnp.maximum(w, 1e-37))
    m = jnp.max(lw, axis=1, keepdims=True)
    e = jnp.exp(lw - m)
    p = e / jnp.sum(e, axis=1, keepdims=True)
    nv_all = w / (nr_ref[0, 0] * p + 1e-8)

    lw2 = jnp.concatenate([lw, lw], axis=1)
    iota_f = lax.broadcasted_iota(jnp.int32, (_B1, _J), 1).astype(jnp.float32)
    inds = [None] * _S
    for g in range(8):
        sg = noise_ref[g] + lw2
        for h in (0, 1):
            sc = sg[:, 64 * h:64 * h + 64]
            mx = jnp.max(sc, axis=1, keepdims=True)
            inds[h * 8 + g] = jnp.min(
                jnp.where(sc == mx, iota_f, float(_J)), axis=1, keepdims=True)
    ind16 = jnp.concatenate(inds, axis=1).astype(jnp.int32)

    nv16 = jnp.take_along_axis(nv_all, ind16, axis=1)
    spx = jnp.take_along_axis(px_ref[...], ind16, axis=1) * nv16
    spy = jnp.take_along_axis(py_ref[...], ind16, axis=1) * nv16
    spz = jnp.take_along_axis(pz_ref[...], ind16, axis=1) * nv16

    rows = pl.program_id(0) * _B1 + lax.broadcasted_iota(jnp.int32, (_B1, _S), 0)
    gi_ref[...] = rows * _S + (ind16 >> 2)
    nv_ref[...] = nv16
    sp_ref[...] = jnp.stack(
        [spx, spy, spz, (ind16 & 3).astype(jnp.float32)], axis=2)


def _stage1(nr, weights, noise, px, py, pz):
    return pl.pallas_call(
        _stage1_body,
        grid=(_R // _B1,),
        in_specs=[
            pl.BlockSpec(memory_space=pltpu.SMEM),
            pl.BlockSpec((_B1, _J), lambda i: (i, 0)),
            pl.BlockSpec((8, _B1, 128), lambda i: (0, i, 0)),
            pl.BlockSpec((_B1, _J), lambda i: (i, 0)),
            pl.BlockSpec((_B1, _J), lambda i: (i, 0)),
            pl.BlockSpec((_B1, _J), lambda i: (i, 0)),
        ],
        out_specs=[
            pl.BlockSpec((_B1, _S), lambda i: (i, 0)),
            pl.BlockSpec((_B1, _S), lambda i: (i, 0)),
            pl.BlockSpec((_B1, _S, _PD + 1), lambda i: (i, 0, 0)),
        ],
        out_shape=[
            jax.ShapeDtypeStruct((_R, _S), jnp.int32),
            jax.ShapeDtypeStruct((_R, _S), jnp.float32),
            jax.ShapeDtypeStruct((_R, _S, _PD + 1), jnp.float32),
        ],
    )(nr, weights, noise, px, py, pz)


_TBLK = 512
_GCHUNK = 512


def _pack_body(x_ref, o_ref):
    ident = (lax.broadcasted_iota(jnp.int32, (128, 128), 0) ==
             lax.broadcasted_iota(jnp.int32, (128, 128), 1)).astype(jnp.float32)
    for q in range(_S):
        for p in range(_TBLK // 128):
            t = x_ref[pl.ds(128 * q, 128), pl.ds(128 * p, 128)]
            tt = lax.dot_general(t, ident, (((0,), (0,)), ((), ())),
                                 precision=lax.Precision.HIGHEST,
                                 preferred_element_type=jnp.float32)
            o_ref[pl.ds(128 * p, 128), q, :] = tt


def _pack(feat_lin):
    return pl.pallas_call(
        _pack_body,
        grid=(_R // _TBLK,),
        in_specs=[pl.BlockSpec((_J * _FD, _TBLK), lambda i: (0, i))],
        out_specs=pl.BlockSpec((_TBLK, _S, 128), lambda i: (i, 0, 0)),
        out_shape=jax.ShapeDtypeStruct((_R, _S, 128), jnp.float32),
    )(feat_lin)


@functools.cache
def _sc_gather_fn():
    @functools.partial(
        pl.kernel,
        out_type=jax.ShapeDtypeStruct((_TOT, 128), jnp.float32),
        mesh=plsc.VectorSubcoreMesh(core_axis_name="c", subcore_axis_name="s"),
        scratch_types=[
            pltpu.VMEM((_GCHUNK,), jnp.int32),
            pltpu.VMEM((_GCHUNK, 128), jnp.float32),
            pltpu.SemaphoreType.DMA,
        ],
        compiler_params=pltpu.CompilerParams(use_tc_tiling_on_sc=False),
    )
    def _sc_gather(gi_hbm, packed_hbm, outf_hbm, idx_v, fbuf, sem_f):
        wid = lax.axis_index("c") * 16 + lax.axis_index("s")
        per_w = _TOT // _NW
        for t in range(per_w // _GCHUNK):
            base = wid * per_w + t * _GCHUNK
            pltpu.sync_copy(gi_hbm.at[pl.ds(base, _GCHUNK)], idx_v)
            pltpu.async_copy(packed_hbm.at[idx_v], fbuf, sem_f).wait()
            pltpu.sync_copy(fbuf, outf_hbm.at[pl.ds(base, _GCHUNK), :])

    return _sc_gather


def _stage3_body(f_ref, sp_ref, nv_ref, o_ref):
    nv = nv_ref[...][:, :, None]
    q = sp_ref[:, :, _PD:_PD + 1].astype(jnp.int32)
    idx = q * _FD + lax.broadcasted_iota(jnp.int32, (_B3, _S, _FD), 2)
    f = jnp.take_along_axis(f_ref[...], idx, axis=2)
    o_ref[...] = jnp.concatenate([nv * f, sp_ref[:, :, 0:_PD]], axis=2)


def _stage3(outf, sp, nv):
    return pl.pallas_call(
        _stage3_body,
        grid=(_R // _B3,),
        in_specs=[
            pl.BlockSpec((_B3, _S, 128), lambda i: (i, 0, 0)),
            pl.BlockSpec((_B3, _S, _PD + 1), lambda i: (i, 0, 0)),
            pl.BlockSpec((_B3, _S), lambda i: (i, 0)),
        ],
        out_specs=pl.BlockSpec((_B3, _S, _FD + _PD), lambda i: (i, 0, 0)),
        out_shape=jax.ShapeDtypeStruct((_R, _S, _FD + _PD), jnp.float32),
    )(outf, sp, nv)


def kernel(weights, points, features, num_resample):
    nr = jnp.asarray(num_resample, jnp.float32).reshape(1, 1)
    px = points[:, :, 0]
    py = points[:, :, 1]
    pz = points[:, :, 2]
    gi, nv, sp = _stage1(nr, weights, jnp.asarray(_NOISE), px, py, pz)
    feat_lin = features.transpose(1, 2, 0).reshape(_J * _FD, _R)
    packed = _pack(feat_lin)
    outf = _sc_gather_fn()(gi.reshape(_TOT), packed.reshape(_TOT, 128))
    return _stage3(outf.reshape(_R, _S, 128), sp, nv)


# speedup vs baseline: 1.0461x; 1.0003x over previous
import functools

import numpy as np

import jax
import jax.numpy as jnp
from jax import lax
from jax.experimental import pallas as pl
from jax.experimental.pallas import tpu as pltpu
from jax.experimental.pallas import tpu_sc as plsc

_R = 16384
_J = 64
_S = 16
_FD = 32
_PD = 3
_TOT = _R * _S


def _gumbel_noise_packed() -> np.ndarray:
    n = _R * _J * _S
    x0 = np.zeros(n, dtype=np.uint32)
    x1 = np.arange(n, dtype=np.uint32)
    ks0 = np.uint32(0)
    ks1 = np.uint32(42)
    ks2 = np.uint32(ks0 ^ ks1 ^ np.uint32(0x1BD11BDA))
    ks = (ks0, ks1, ks2)
    rot = (13, 15, 26, 6, 17, 29, 16, 24)

    x0 += ks0
    x1 += ks1
    for g in range(5):
        for r in (rot[0:4] if g % 2 == 0 else rot[4:8]):
            x0 += x1
            x1 = ((x1 << np.uint32(r)) | (x1 >> np.uint32(32 - r)))
            x1 ^= x0
        x0 += ks[(g + 1) % 3]
        x1 += ks[(g + 2) % 3] + np.uint32(g + 1)
    bits = x0 ^ x1

    tiny = np.float32(np.finfo(np.float32).tiny)
    fb = (bits >> np.uint32(9)) | np.uint32(0x3F800000)
    f = fb.view(np.float32) - np.float32(1.0)
    u = np.maximum(tiny, f * (np.float32(1.0) - tiny) + tiny)
    g32 = (-np.log(-np.log(u))).astype(np.float32)

    g3 = g32.reshape(_R, _J, _S)
    packed = np.empty((8, _R, 128), dtype=np.float32)
    for h in (0, 1):
        packed[:, :, 64 * h:64 * h + 64] = np.ascontiguousarray(
            g3[:, :, 8 * h:8 * h + 8].transpose(2, 0, 1))
    return packed


_NOISE = _gumbel_noise_packed()

_B1 = 1024
_B3 = 256
_CHUNK = 2048
_NW = 32


def _stage1_body(nr_ref, w_ref, noise_ref, px_ref, py_ref, pz_ref,
                 gi_ref, nv_ref, sp_ref):
    w = w_ref[...]
    lw = jnp.log(jnp.maximum(w, 1e-37))
    m = jnp.max(lw, axis=1, keepdims=True)
    e = jnp.exp(lw - m)
    p = e / jnp.sum(e, axis=1, keepdims=True)
    nv_all = w / (nr_ref[0, 0] * p + 1e-8)

    lw2 = jnp.concatenate([lw, lw], axis=1)
    iota_f = lax.broadcasted_iota(jnp.int32, (_B1, _J), 1).astype(jnp.float32)
    inds = [None] * _S
    for g in range(8):
        sg = noise_ref[g] + lw2
        for h in (0, 1):
            sc = sg[:, 64 * h:64 * h + 64]
            mx = jnp.max(sc, axis=1, keepdims=True)
            inds[h * 8 + g] = jnp.min(
                jnp.where(sc == mx, iota_f, float(_J)), axis=1, keepdims=True)
    ind16 = jnp.concatenate(inds, axis=1).astype(jnp.int32)

    nv16 = jnp.take_along_axis(nv_all, ind16, axis=1)
    spx = jnp.take_along_axis(px_ref[...], ind16, axis=1) * nv16
    spy = jnp.take_along_axis(py_ref[...], ind16, axis=1) * nv16
    spz = jnp.take_along_axis(pz_ref[...], ind16, axis=1) * nv16

    rows = pl.program_id(0) * _B1 + lax.broadcasted_iota(jnp.int32, (_B1, _S), 0)
    gi_ref[...] = rows * _S + (ind16 >> 2)
    nv_ref[...] = nv16
    sp_ref[...] = jnp.stack(
        [spx, spy, spz, (ind16 & 3).astype(jnp.float32)], axis=2)


def _stage1(nr, weights, noise, px, py, pz):
    return pl.pallas_call(
        _stage1_body,
        grid=(_R // _B1,),
        in_specs=[
            pl.BlockSpec(memory_space=pltpu.SMEM),
            pl.BlockSpec((_B1, _J), lambda i: (i, 0)),
            pl.BlockSpec((8, _B1, 128), lambda i: (0, i, 0)),
            pl.BlockSpec((_B1, _J), lambda i: (i, 0)),
            pl.BlockSpec((_B1, _J), lambda i: (i, 0)),
            pl.BlockSpec((_B1, _J), lambda i: (i, 0)),
        ],
        out_specs=[
            pl.BlockSpec((_B1, _S), lambda i: (i, 0)),
            pl.BlockSpec((_B1, _S), lambda i: (i, 0)),
            pl.BlockSpec((_B1, _S, _PD + 1), lambda i: (i, 0, 0)),
        ],
        out_shape=[
            jax.ShapeDtypeStruct((_R, _S), jnp.int32),
            jax.ShapeDtypeStruct((_R, _S), jnp.float32),
            jax.ShapeDtypeStruct((_R, _S, _PD + 1), jnp.float32),
        ],
    )(nr, weights, noise, px, py, pz)


_TBLK = 512
_GCHUNK = 512


def _pack_body(x_ref, o_ref):
    ident = (lax.broadcasted_iota(jnp.int32, (128, 128), 0) ==
             lax.broadcasted_iota(jnp.int32, (128, 128), 1)).astype(jnp.float32)
    for q in range(_S):
        for p in range(_TBLK // 128):
            t = x_ref[pl.ds(128 * q, 128), pl.ds(128 * p, 128)]
            tt = lax.dot_general(t, ident, (((0,), (0,)), ((), ())),
                                 precision=lax.Precision.HIGHEST,
                                 preferred_element_type=jnp.float32)
            o_ref[pl.ds(128 * p, 128), q, :] = tt


def _pack(feat_lin):
    return pl.pallas_call(
        _pack_body,
        grid=(_R // _TBLK,),
        in_specs=[pl.BlockSpec((_J * _FD, _TBLK), lambda i: (0, i))],
        out_specs=pl.BlockSpec((_TBLK, _S, 128), lambda i: (i, 0, 0)),
        out_shape=jax.ShapeDtypeStruct((_R, _S, 128), jnp.float32),
    )(feat_lin)


@functools.cache
def _sc_gather_fn():
    @functools.partial(
        pl.kernel,
        out_type=jax.ShapeDtypeStruct((_TOT, 128), jnp.float32),
        mesh=plsc.VectorSubcoreMesh(core_axis_name="c", subcore_axis_name="s"),
        scratch_types=[
            pltpu.VMEM((_GCHUNK,), jnp.int32),
            pltpu.VMEM((_GCHUNK, 128), jnp.float32),
            pltpu.SemaphoreType.DMA,
        ],
        compiler_params=pltpu.CompilerParams(use_tc_tiling_on_sc=False),
    )
    def _sc_gather(gi_hbm, packed_hbm, outf_hbm, idx_v, fbuf, sem_f):
        wid = lax.axis_index("c") * 16 + lax.axis_index("s")
        per_w = _TOT // _NW
        for t in range(per_w // _GCHUNK):
            base = wid * per_w + t * _GCHUNK
            pltpu.sync_copy(gi_hbm.at[pl.ds(base, _GCHUNK)], idx_v)
            pltpu.async_copy(packed_hbm.at[idx_v], fbuf, sem_f).wait()
            pltpu.sync_copy(fbuf, outf_hbm.at[pl.ds(base, _GCHUNK), :])

    return _sc_gather


def _stage3_body(f_ref, sp_ref, nv_ref, o_ref):
    nv = nv_ref[...][:, :, None]
    q = sp_ref[:, :, _PD:_PD + 1].astype(jnp.int32)
    idx = q * _FD + lax.broadcasted_iota(jnp.int32, (_B3, _S, _FD), 2)
    f = jnp.take_along_axis(f_ref[...], idx, axis=2)
    o_ref[...] = jnp.concatenate([nv * f, sp_ref[:, :, 0:_PD]], axis=2)


def _stage3(outf, sp, nv):
    return pl.pallas_call(
        _stage3_body,
        grid=(_R // _B3,),
        in_specs=[
            pl.BlockSpec((_B3, _S, 128), lambda i: (i, 0, 0)),
            pl.BlockSpec((_B3, _S, _PD + 1), lambda i: (i, 0, 0)),
            pl.BlockSpec((_B3, _S), lambda i: (i, 0)),
        ],
        out_specs=pl.BlockSpec((_B3, _S, _FD + _PD), lambda i: (i, 0, 0)),
        out_shape=jax.ShapeDtypeStruct((_R, _S, _FD + _PD), jnp.float32),
    )(outf, sp, nv)


def kernel(weights, points, features, num_resample):
    nr = jnp.asarray(num_resample, jnp.float32).reshape(1, 1)
    px = points[:, :, 0]
    py = points[:, :, 1]
    pz = points[:, :, 2]
    gi, nv, sp = _stage1(nr, weights, jnp.asarray(_NOISE), px, py, pz)
    feat_lin = features.transpose(1, 2, 0).reshape(_J * _FD, _R)
    packed = _pack(feat_lin)
    outf = _sc_gather_fn()(gi.reshape(_TOT), packed.reshape(_TOT, 128))
    return _stage3(outf.reshape(_R, _S, 128), sp, nv)
